# baseline (speedup 1.0000x reference)
import jax
import jax.numpy as jnp
from jax import lax
from jax.experimental import pallas as pl
from jax.experimental.pallas import tpu as pltpu
from jax.experimental.pallas import tpu_sc as plsc

MAX_POS = 77
HIDDEN = 768
LANES = 16
NCORES = 2
NSUB = 16
NW = NCORES * NSUB

CB = 16
COLS = HIDDEN // LANES
NBUF = 7
AHEAD = 4
NPOS = 16


def _emb_body(ids_hbm, tok_hbm, pos_hbm, out_hbm, *scr):
    idx_v, pos_v = scr[0], scr[1]
    bufs = scr[2:2 + NBUF]
    gsems = scr[2 + NBUF:2 + 2 * NBUF]
    osems = scr[2 + 2 * NBUF:2 + 3 * NBUF]
    psem = scr[2 + 3 * NBUF]
    w = lax.axis_index("s") * NCORES + lax.axis_index("c")
    nch = ids_hbm.shape[1]
    batch = out_hbm.shape[0] // MAX_POS

    pltpu.sync_copy(ids_hbm.at[w], idx_v)
    pbase = lax.min(lax.div(w * nch * CB, batch), MAX_POS - NPOS)
    pidx = pbase + lax.iota(jnp.int32, LANES)
    pltpu.async_copy(pos_hbm.at[pidx], pos_v, psem).wait()

    def gather_desc(c, buf, sem):
        return pltpu.make_async_copy(tok_hbm.at[idx_v.at[c]], buf, sem)

    def out_desc(c, buf, sem):
        row0 = (w * nch + c) * CB
        return pltpu.make_async_copy(buf, out_hbm.at[pl.ds(row0, CB)], sem)

    def add_pos(c, buf):
        srow = lax.div((w * nch + c) * CB, batch) - pbase
        pos_regs = [pos_v[srow, pl.ds(j * LANES, LANES)] for j in range(COLS)]

        def rows(r, _):
            for u in range(2):
                for j in range(COLS):
                    plsc.addupdate(
                        buf.at[2 * r + u, pl.ds(j * LANES, LANES)], pos_regs[j])
            return 0

        lax.fori_loop(0, CB // 2, rows, 0)

    def maybe(cond, fn):
        if isinstance(cond, bool):
            if cond:
                fn()
        else:
            pl.when(cond)(fn)

    def step(c, k):
        kr = (k + AHEAD) % NBUF
        maybe(c >= NBUF - AHEAD,
              lambda: out_desc(c - (NBUF - AHEAD), bufs[kr], osems[kr]).wait())
        maybe(c <= nch - 1 - AHEAD,
              lambda: gather_desc(c + AHEAD, bufs[kr], gsems[kr]).start())

        gather_desc(c, bufs[k], gsems[k]).wait()
        add_pos(c, bufs[k])
        out_desc(c, bufs[k], osems[k]).start()

    for k in range(AHEAD):
        gather_desc(k, bufs[k], gsems[k]).start()

    def ring(t, _):
        for k in range(NBUF):
            step(NBUF * t + k, k)
        return 0

    lax.fori_loop(0, nch // NBUF, ring, 0)
    for c in range(nch - nch % NBUF, nch):
        step(c, c % NBUF)

    for c in range(nch - (NBUF - AHEAD), nch):
        out_desc(c, bufs[c % NBUF], osems[c % NBUF]).wait()


def kernel(input_ids, token_embedding, position_embedding):
    b, s = input_ids.shape
    rows = b * s
    assert rows % (NW * CB) == 0 and b % CB == 0
    nch = rows // (NW * CB)
    ids2 = input_ids.T.astype(jnp.int32).reshape(NW, nch, CB)
    run = pl.kernel(
        _emb_body,
        out_type=jax.ShapeDtypeStruct((rows, HIDDEN), jnp.float32),
        mesh=plsc.VectorSubcoreMesh(core_axis_name="c", subcore_axis_name="s"),
        scratch_types=[
            pltpu.VMEM((nch, CB), jnp.int32),
            pltpu.VMEM((NPOS, HIDDEN), jnp.float32),
        ] + [pltpu.VMEM((CB, HIDDEN), jnp.float32)] * NBUF
          + [pltpu.SemaphoreType.DMA] * (2 * NBUF + 1),
    )
    out = run(ids2, token_embedding, position_embedding)
    return jnp.swapaxes(out.reshape(s, b, HIDDEN), 0, 1)


# speedup vs baseline: 1.0056x; 1.0056x over previous
import jax
import jax.numpy as jnp
from jax import lax
from jax.experimental import pallas as pl
from jax.experimental.pallas import tpu as pltpu
from jax.experimental.pallas import tpu_sc as plsc

MAX_POS = 77
HIDDEN = 768
LANES = 16
NCORES = 2
NSUB = 16
NW = NCORES * NSUB

CB = 32
COLS = HIDDEN // LANES
NBUF = 4
AHEAD = 2
NPOS = 16


def _emb_body(ids_hbm, tok_hbm, pos_hbm, out_hbm, *scr):
    idx_v, pos_v = scr[0], scr[1]
    bufs = scr[2:2 + NBUF]
    gsems = scr[2 + NBUF:2 + 2 * NBUF]
    osems = scr[2 + 2 * NBUF:2 + 3 * NBUF]
    psem = scr[2 + 3 * NBUF]
    w = lax.axis_index("s") * NCORES + lax.axis_index("c")
    nch = ids_hbm.shape[0] // (NW * CB)
    batch = out_hbm.shape[0] // MAX_POS

    pltpu.sync_copy(ids_hbm.at[pl.ds(w * nch * CB, nch * CB)], idx_v)
    pbase = lax.min(lax.div(w * nch * CB, batch), MAX_POS - NPOS)
    pidx = pbase + lax.iota(jnp.int32, LANES)
    pltpu.async_copy(pos_hbm.at[pidx], pos_v, psem).wait()

    def gather_desc(c, buf, sem):
        return pltpu.make_async_copy(
            tok_hbm.at[idx_v.at[pl.ds(c * CB, CB)]], buf, sem)

    def out_desc(c, buf, sem):
        row0 = (w * nch + c) * CB
        return pltpu.make_async_copy(buf, out_hbm.at[pl.ds(row0, CB)], sem)

    def add_pos(c, buf):
        srow = lax.div((w * nch + c) * CB, batch) - pbase
        pos_regs = [pos_v[srow, pl.ds(j * LANES, LANES)] for j in range(COLS)]

        def rows(r, _):
            for u in range(2):
                for j in range(COLS):
                    plsc.addupdate(
                        buf.at[2 * r + u, pl.ds(j * LANES, LANES)], pos_regs[j])
            return 0

        lax.fori_loop(0, CB // 2, rows, 0)

    def maybe(cond, fn):
        if isinstance(cond, bool):
            if cond:
                fn()
        else:
            pl.when(cond)(fn)

    def step(c, k):
        kr = (k + AHEAD) % NBUF
        maybe(c >= NBUF - AHEAD,
              lambda: out_desc(c - (NBUF - AHEAD), bufs[kr], osems[kr]).wait())
        maybe(c <= nch - 1 - AHEAD,
              lambda: gather_desc(c + AHEAD, bufs[kr], gsems[kr]).start())

        gather_desc(c, bufs[k], gsems[k]).wait()
        add_pos(c, bufs[k])
        out_desc(c, bufs[k], osems[k]).start()

    for k in range(AHEAD):
        gather_desc(k, bufs[k], gsems[k]).start()

    def ring(t, _):
        for k in range(NBUF):
            step(NBUF * t + k, k)
        return 0

    lax.fori_loop(0, nch // NBUF, ring, 0)
    for c in range(nch - nch % NBUF, nch):
        step(c, c % NBUF)

    for c in range(nch - (NBUF - AHEAD), nch):
        out_desc(c, bufs[c % NBUF], osems[c % NBUF]).wait()


def kernel(input_ids, token_embedding, position_embedding):
    b, s = input_ids.shape
    rows = b * s
    assert rows % (NW * CB) == 0 and b % CB == 0
    nch = rows // (NW * CB)
    ids2 = input_ids.T.astype(jnp.int32).reshape(rows)
    run = pl.kernel(
        _emb_body,
        out_type=jax.ShapeDtypeStruct((rows, HIDDEN), jnp.float32),
        mesh=plsc.VectorSubcoreMesh(core_axis_name="c", subcore_axis_name="s"),
        scratch_types=[
            pltpu.VMEM((nch * CB,), jnp.int32),
            pltpu.VMEM((NPOS, HIDDEN), jnp.float32),
        ] + [pltpu.VMEM((CB, HIDDEN), jnp.float32)] * NBUF
          + [pltpu.SemaphoreType.DMA] * (2 * NBUF + 1),
    )
    out = run(ids2, token_embedding, position_embedding)
    return jnp.swapaxes(out.reshape(s, b, HIDDEN), 0, 1)
